# Initial kernel scaffold; baseline (speedup 1.0000x reference)
import jax
import jax.numpy as jnp
from jax.experimental import pallas as pl


def kernel(data, selection, phase):
    raise NotImplementedError("write your pallas kernel here")


# speedup vs baseline: 1.0648x; 1.0648x over previous
import jax
import jax.numpy as jnp
import numpy as np
from jax.experimental import pallas as pl
from jax.experimental.pallas import tpu as pltpu

_F_SAMPLE = 250.0
_FREQ = 0.5
_AMPLITUDE = 0.05
_BLOCK_ROWS = 256


def _osc_kernel(sel_ref, phase_ref, data_ref, out_ref):
    i = pl.program_id(0)
    br, t_len = data_ref.shape
    rows = i * br + jax.lax.broadcasted_iota(jnp.int32, (br, 1), 0)
    sel = sel_ref[0, :]
    hit = (rows == sel[None, :]).any(axis=1, keepdims=True)
    col = jax.lax.broadcasted_iota(jnp.int32, (1, t_len), 1).astype(jnp.float32)
    step = (t_len / _F_SAMPLE) / (t_len - 1)
    osc = _AMPLITUDE * jnp.sin(
        (2.0 * np.pi * _FREQ * step) * col + phase_ref[0]
    )
    out_ref[...] = data_ref[...] + jnp.where(hit, osc, 0.0)


def kernel(data, selection, phase):
    n_ts, t_len = data.shape
    sel2 = selection.astype(jnp.int32).reshape(1, -1)
    phase_arr = jnp.reshape(phase, (1,)).astype(jnp.float32)
    grid = (n_ts // _BLOCK_ROWS,)
    return pl.pallas_call(
        _osc_kernel,
        grid=grid,
        in_specs=[
            pl.BlockSpec((1, sel2.shape[1]), lambda i: (0, 0)),
            pl.BlockSpec(memory_space=pltpu.SMEM),
            pl.BlockSpec((_BLOCK_ROWS, t_len), lambda i: (i, 0)),
        ],
        out_specs=pl.BlockSpec((_BLOCK_ROWS, t_len), lambda i: (i, 0)),
        out_shape=jax.ShapeDtypeStruct((n_ts, t_len), jnp.float32),
        compiler_params=pltpu.CompilerParams(
            dimension_semantics=("arbitrary",),
        ),
    )(sel2, phase_arr, data)


# speedup vs baseline: 1.0978x; 1.0310x over previous
import jax
import jax.numpy as jnp
import numpy as np
from jax.experimental import pallas as pl
from jax.experimental.pallas import tpu as pltpu

_F_SAMPLE = 250.0
_FREQ = 0.5
_AMPLITUDE = 0.05
_BLOCK_ROWS = 512


def _osc_kernel(sel_ref, phase_ref, data_ref, out_ref):
    i = pl.program_id(0)
    br, t_len = data_ref.shape
    rows = i * br + jax.lax.broadcasted_iota(jnp.int32, (br, 1), 0)
    sel = sel_ref[0, :]
    hit = (rows == sel[None, :]).any(axis=1, keepdims=True)
    col = jax.lax.broadcasted_iota(jnp.int32, (1, t_len), 1).astype(jnp.float32)
    step = (t_len / _F_SAMPLE) / (t_len - 1)
    osc = _AMPLITUDE * jnp.sin(
        (2.0 * np.pi * _FREQ * step) * col + phase_ref[0]
    )
    out_ref[...] = data_ref[...] + jnp.where(hit, osc, 0.0)


def kernel(data, selection, phase):
    n_ts, t_len = data.shape
    sel2 = selection.astype(jnp.int32).reshape(1, -1)
    phase_arr = jnp.reshape(phase, (1,)).astype(jnp.float32)
    grid = (n_ts // _BLOCK_ROWS,)
    return pl.pallas_call(
        _osc_kernel,
        grid=grid,
        in_specs=[
            pl.BlockSpec((1, sel2.shape[1]), lambda i: (0, 0)),
            pl.BlockSpec(memory_space=pltpu.SMEM),
            pl.BlockSpec((_BLOCK_ROWS, t_len), lambda i: (i, 0)),
        ],
        out_specs=pl.BlockSpec((_BLOCK_ROWS, t_len), lambda i: (i, 0)),
        out_shape=jax.ShapeDtypeStruct((n_ts, t_len), jnp.float32),
        compiler_params=pltpu.CompilerParams(
            dimension_semantics=("arbitrary",),
        ),
    )(sel2, phase_arr, data)
